# baseline (speedup 1.0000x reference)
import functools

import jax
import jax.numpy as jnp
from jax import lax
from jax.experimental import pallas as pl
from jax.experimental.pallas import tpu as pltpu
from jax.experimental.pallas import tpu_sc as plsc

B = 16384
IN = 64
OUT = 64
GRID = 20
L = 23
LP = 24
NLANE = 16

NC = 2
NS = 16
NW = NC * NS
SPW = B // NW
CHUNK = 64
NCHUNK = SPW // CHUNK


def _body(x_hbm, c2_hbm, out_hbm, table_v, xbuf, obuf):
    wid = lax.axis_index("s") * NC + lax.axis_index("c")
    base = wid * SPW

    pltpu.sync_copy(c2_hbm, table_v)

    def chunk_body(ci, _):
        row0 = base + ci * CHUNK
        pltpu.sync_copy(x_hbm.at[pl.ds(row0, CHUNK)], xbuf)

        def sample_body(b, _):
            accs = [jnp.zeros((NLANE,), jnp.float32)] * (OUT // NLANE)
            for jc in range(IN // NLANE):
                xv = xbuf[b, pl.ds(jc * NLANE, NLANE)]
                sig = 1.0 / (1.0 + jnp.exp(-xv))
                idx = sig * float(GRID - 1)
                k = idx.astype(jnp.int32)
                w1v = idx - k.astype(jnp.float32)
                k0 = jnp.clip(k, 0, L - 1)
                k1 = jnp.clip(k + 1, 0, L - 1)
                j = lax.iota(jnp.int32, NLANE) + jc * NLANE
                off0v = (j * LP + k0) * OUT
                off1v = (j * LP + k1) * OUT
                for i in range(NLANE):
                    o0 = off0v[i]
                    o1 = off1v[i]
                    w1 = w1v[i]
                    w0 = 1.0 - w1
                    for oc in range(OUT // NLANE):
                        accs[oc] = accs[oc] + (
                            table_v[pl.ds(o0 + oc * NLANE, NLANE)] * w0
                            + table_v[pl.ds(o1 + oc * NLANE, NLANE)] * w1
                        )
            for oc in range(OUT // NLANE):
                obuf[b, pl.ds(oc * NLANE, NLANE)] = accs[oc]
            return ()

        lax.fori_loop(0, CHUNK, sample_body, ())
        pltpu.sync_copy(obuf, out_hbm.at[pl.ds(row0, CHUNK)])
        return ()

    lax.fori_loop(0, NCHUNK, chunk_body, ())


@jax.jit
def kernel(x, coeffs):
    c2 = jnp.transpose(coeffs, (1, 2, 0))
    c2 = jnp.pad(c2, ((0, 0), (0, LP - L), (0, 0)))
    c2 = c2.reshape(IN * LP * OUT)

    mesh = plsc.VectorSubcoreMesh(core_axis_name="c", subcore_axis_name="s")
    run = functools.partial(
        pl.kernel,
        out_type=jax.ShapeDtypeStruct((B, OUT), jnp.float32),
        mesh=mesh,
        scratch_types=[
            pltpu.VMEM((IN * LP * OUT,), jnp.float32),
            pltpu.VMEM((CHUNK, IN), jnp.float32),
            pltpu.VMEM((CHUNK, OUT), jnp.float32),
        ],
    )(_body)
    return run(x, c2)


# speedup vs baseline: 1.2440x; 1.2440x over previous
import functools

import jax
import jax.numpy as jnp
from jax import lax
from jax.experimental import pallas as pl
from jax.experimental.pallas import tpu as pltpu
from jax.experimental.pallas import tpu_sc as plsc

B = 16384
IN = 64
OUT = 64
GRID = 20
L = 23
LP = 24
NLANE = 16

NC = 2
NS = 16
NW = NC * NS
SPW = B // NW
CHUNK = 64
NCHUNK = SPW // CHUNK


def _body(x_hbm, c2_hbm, out_hbm, table_v, xbuf, obuf):
    wid = lax.axis_index("s") * NC + lax.axis_index("c")
    base = wid * SPW

    pltpu.sync_copy(c2_hbm, table_v)

    def chunk_body(ci, _):
        row0 = base + ci * CHUNK
        pltpu.sync_copy(x_hbm.at[pl.ds(row0, CHUNK)], xbuf)

        def sample_body(b, _):
            accs = [jnp.zeros((NLANE,), jnp.float32)] * (OUT // NLANE)
            for jc in range(IN // NLANE):
                xv = xbuf[b, pl.ds(jc * NLANE, NLANE)]
                sig = 1.0 / (1.0 + jnp.exp(-xv))
                idx = sig * float(GRID - 1)
                k = idx.astype(jnp.int32)
                w1v = idx - k.astype(jnp.float32)
                k0 = jnp.clip(k, 0, L - 1)
                k1 = jnp.clip(k + 1, 0, L - 1)
                j = lax.iota(jnp.int32, NLANE) + jc * NLANE
                off0v = (j * LP + k0) * OUT
                off1v = (j * LP + k1) * OUT
                for i in range(NLANE):
                    o0 = off0v[i]
                    o1 = off1v[i]
                    w1 = w1v[i]
                    w0 = 1.0 - w1
                    for oc in range(OUT // NLANE):
                        accs[oc] = (
                            accs[oc]
                            + table_v[pl.ds(o0 + oc * NLANE, NLANE)] * w0
                            + table_v[pl.ds(o1 + oc * NLANE, NLANE)] * w1
                        )
            for oc in range(OUT // NLANE):
                obuf[b, pl.ds(oc * NLANE, NLANE)] = accs[oc]
            return ()

        lax.fori_loop(0, CHUNK, sample_body, ())
        pltpu.sync_copy(obuf, out_hbm.at[pl.ds(row0, CHUNK)])
        return ()

    lax.fori_loop(0, NCHUNK, chunk_body, ())


@jax.jit
def kernel(x, coeffs):
    c2 = jnp.transpose(coeffs, (1, 2, 0))
    c2 = jnp.pad(c2, ((0, 0), (0, LP - L), (0, 0)))
    c2 = c2.reshape(IN * LP * OUT)

    mesh = plsc.VectorSubcoreMesh(core_axis_name="c", subcore_axis_name="s")
    run = functools.partial(
        pl.kernel,
        out_type=jax.ShapeDtypeStruct((B, OUT), jnp.float32),
        mesh=mesh,
        scratch_types=[
            pltpu.VMEM((IN * LP * OUT,), jnp.float32),
            pltpu.VMEM((CHUNK, IN), jnp.float32),
            pltpu.VMEM((CHUNK, OUT), jnp.float32),
        ],
    )(_body)
    return run(x, c2)
